# baseline (speedup 1.0000x reference)
import functools

import jax
import jax.numpy as jnp
from jax import lax
from jax.experimental import pallas as pl
from jax.experimental.pallas import tpu as pltpu
from jax.experimental.pallas import tpu_sc as plsc

_NUM_CORES = 2
_NUM_SUBCORES = 16
_NUM_WORKERS = _NUM_CORES * _NUM_SUBCORES


@functools.lru_cache(maxsize=None)
def _build(B, V, D, n_chunks):
    b_per_w = B // _NUM_WORKERS
    chunk = b_per_w // n_chunks
    mesh = plsc.VectorSubcoreMesh(core_axis_name="c", subcore_axis_name="s")

    @functools.partial(
        pl.kernel,
        mesh=mesh,
        out_type=jax.ShapeDtypeStruct((B, D), jnp.float32),
        scratch_types=[
            pltpu.VMEM((b_per_w,), jnp.int32),
            pltpu.VMEM((n_chunks, chunk, D), jnp.float32),
            pltpu.SemaphoreType.DMA,
        ]
        + [pltpu.SemaphoreType.DMA] * (2 * n_chunks),
        compiler_params=pltpu.CompilerParams(use_tc_tiling_on_sc=False),
    )
    def gather_kernel(idx_hbm, table_hbm, out_hbm, idx_v, rows_v, isem, *sems):
        gsems, osems = sems[:n_chunks], sems[n_chunks:]
        wid = lax.axis_index("s") * _NUM_CORES + lax.axis_index("c")
        base = wid * b_per_w
        pltpu.async_copy(idx_hbm.at[pl.ds(base, b_per_w)], idx_v, isem).wait()

        gathers = []
        for c in range(n_chunks):
            gathers.append(
                pltpu.async_copy(
                    table_hbm.at[idx_v.at[pl.ds(c * chunk, chunk)]],
                    rows_v.at[c],
                    gsems[c],
                )
            )
        outs = []
        for c in range(n_chunks):
            gathers[c].wait()
            outs.append(
                pltpu.async_copy(
                    rows_v.at[c],
                    out_hbm.at[pl.ds(base + c * chunk, chunk)],
                    osems[c],
                )
            )
        for c in range(n_chunks):
            outs[c].wait()

    return gather_kernel


def kernel(token_ids, E):
    B = token_ids.shape[0]
    V, D = E.shape
    idx = token_ids.astype(jnp.int32)
    return _build(B, V, D, 4)(idx, E)
